# Initial kernel scaffold; baseline (speedup 1.0000x reference)
import functools

import jax
import jax.numpy as jnp
from jax import lax
from jax.experimental import pallas as pl
from jax.experimental.pallas import tpu as pltpu
from jax.experimental.pallas import tpu_sc as plsc

NC = 2
NS = 16
NW = NC * NS
K = 80


def _sc_mesh():
    return plsc.VectorSubcoreMesh(core_axis_name="c", subcore_axis_name="s")


def _make_deg_kernel(n, ch):
    rows = n // NS

    @functools.partial(
        pl.kernel,
        mesh=_sc_mesh(),
        out_type=jax.ShapeDtypeStruct((NC, 2, n, 16), jnp.float32),
        scratch_types=[
            pltpu.VMEM((ch, K), jnp.int32),
            pltpu.VMEM((ch, K), jnp.int32),
            pltpu.VMEM((K, 16), jnp.float32),
            pltpu.VMEM_SHARED((n, 16), jnp.float32),
            pltpu.VMEM_SHARED((n, 16), jnp.float32),
        ],
    )
    def deg_kernel(src_h, dst_h, ones_h, zeros_h, out_h, src_v, dst_v,
                   ones_v, acc_o, acc_i):
        cid = lax.axis_index("c")
        sid = lax.axis_index("s")
        wid = sid * NC + cid
        r0 = sid * rows
        pltpu.sync_copy(src_h.at[wid], src_v)
        pltpu.sync_copy(dst_h.at[wid], dst_v)
        pltpu.sync_copy(ones_h, ones_v)
        pltpu.sync_copy(zeros_h, acc_o.at[pl.ds(r0, rows)])
        pltpu.sync_copy(zeros_h, acc_i.at[pl.ds(r0, rows)])
        plsc.subcore_barrier()

        def body(j, carry):
            pltpu.sync_copy(ones_v, acc_o.at[src_v.at[j]], add=True)
            pltpu.sync_copy(ones_v, acc_i.at[dst_v.at[j]], add=True)
            return carry

        lax.fori_loop(0, ch, body, 0)
        plsc.subcore_barrier()
        pltpu.sync_copy(acc_o.at[pl.ds(r0, rows)],
                        out_h.at[cid, 0, pl.ds(r0, rows)])
        pltpu.sync_copy(acc_i.at[pl.ds(r0, rows)],
                        out_h.at[cid, 1, pl.ds(r0, rows)])

    return deg_kernel


def _make_agg_kernel(n, d, ch):
    rows = n // NS

    @functools.partial(
        pl.kernel,
        mesh=_sc_mesh(),
        out_type=jax.ShapeDtypeStruct((NC, n, d), jnp.float32),
        scratch_types=[
            pltpu.VMEM((ch, K), jnp.int32),
            pltpu.VMEM((ch, K), jnp.int32),
            pltpu.VMEM((K, d), jnp.float32),
            pltpu.VMEM_SHARED((n, d), jnp.float32),
            pltpu.SemaphoreType.DMA,
        ],
    )
    def agg_kernel(h_h, src_h, dst_h, zeros_h, out_h, src_v, dst_v,
                   rows_v, acc, sem):
        cid = lax.axis_index("c")
        sid = lax.axis_index("s")
        wid = sid * NC + cid
        r0 = sid * rows
        pltpu.sync_copy(src_h.at[wid], src_v)
        pltpu.sync_copy(dst_h.at[wid], dst_v)
        pltpu.sync_copy(zeros_h, acc.at[pl.ds(r0, rows)])
        plsc.subcore_barrier()

        def body(j, carry):
            pltpu.async_copy(h_h.at[src_v.at[j]], rows_v, sem).wait()
            pltpu.sync_copy(rows_v, acc.at[dst_v.at[j]], add=True)
            return carry

        lax.fori_loop(0, ch, body, 0)
        plsc.subcore_barrier()
        pltpu.sync_copy(acc.at[pl.ds(r0, rows)],
                        out_h.at[cid, pl.ds(r0, rows)])

    return agg_kernel


def _norm_mm_tc(deg, x, w):
    n, d = x.shape
    h_dim = w.shape[1]

    def body(deg_ref, x_ref, w_ref, ns_ref, nd_ref, h_ref):
        dout = deg_ref[0, 0, :, 0] + deg_ref[1, 0, :, 0]
        din = deg_ref[0, 1, :, 0] + deg_ref[1, 1, :, 0]
        ns = jnp.where(dout > 0, lax.rsqrt(jnp.maximum(dout, 1.0)), 0.0)
        nd = jnp.where(din > 0, lax.rsqrt(jnp.maximum(din, 1.0)), 0.0)
        ns_ref[...] = ns
        nd_ref[...] = nd
        h_ref[...] = jnp.dot(x_ref[...] * ns[:, None], w_ref[...],
                             preferred_element_type=jnp.float32)

    return pl.pallas_call(
        body,
        out_shape=(
            jax.ShapeDtypeStruct((n,), jnp.float32),
            jax.ShapeDtypeStruct((n,), jnp.float32),
            jax.ShapeDtypeStruct((n, h_dim), jnp.float32),
        ),
    )(deg, x, w)


def _post_pre_tc(p, nd, b, g, bt, ns, w):
    n = p.shape[1]
    h_dim = w.shape[1]

    def body(p_ref, nd_ref, b_ref, g_ref, bt_ref, ns_ref, w_ref, h_ref):
        a = ((p_ref[0] + p_ref[1]) * nd_ref[...][:, None] + b_ref[...])
        mean = jnp.mean(a, axis=0)
        var = jnp.mean((a - mean) ** 2, axis=0)
        y = g_ref[...] * (a - mean) * lax.rsqrt(var + 1e-5) + bt_ref[...]
        y = jnp.maximum(y, 0.0)
        h_ref[...] = jnp.dot(y * ns_ref[...][:, None], w_ref[...],
                             preferred_element_type=jnp.float32)

    return pl.pallas_call(
        body,
        out_shape=jax.ShapeDtypeStruct((n, h_dim), jnp.float32),
    )(p, nd, b, g, bt, ns, w)


def _post_pre_nomm_tc(p, nd, b, g, bt, ns):
    n, d = p.shape[1], p.shape[2]

    def body(p_ref, nd_ref, b_ref, g_ref, bt_ref, ns_ref, t_ref):
        a = ((p_ref[0] + p_ref[1]) * nd_ref[...][:, None] + b_ref[...])
        mean = jnp.mean(a, axis=0)
        var = jnp.mean((a - mean) ** 2, axis=0)
        y = g_ref[...] * (a - mean) * lax.rsqrt(var + 1e-5) + bt_ref[...]
        y = jnp.maximum(y, 0.0)
        t_ref[...] = y * ns_ref[...][:, None]

    return pl.pallas_call(
        body,
        out_shape=jax.ShapeDtypeStruct((n, d), jnp.float32),
    )(p, nd, b, g, bt, ns)


def _final_tc(p, nd, w, b):
    n = p.shape[1]
    c = w.shape[1]

    def body(p_ref, nd_ref, w_ref, b_ref, o_ref):
        agg = p_ref[0] + p_ref[1]
        o_ref[...] = (jnp.dot(agg, w_ref[...],
                              preferred_element_type=jnp.float32)
                      * nd_ref[...][:, None] + b_ref[...])

    return pl.pallas_call(
        body,
        out_shape=jax.ShapeDtypeStruct((n, c), jnp.float32),
    )(p, nd, w, b)


def kernel(in_feat, edge_index, W1, b1, g1, bt1, W2, b2, g2, bt2, W3, b3):
    n, d = in_feat.shape
    e = edge_index.shape[1]
    assert e % (NW * K) == 0 and n % NS == 0
    ch = e // (NW * K)

    src3 = edge_index[0].reshape(NW, ch, K)
    dst3 = edge_index[1].reshape(NW, ch, K)
    ones16 = jnp.ones((K, 16), jnp.float32)
    zeros16 = jnp.zeros((n // NS, 16), jnp.float32)
    zerosd = jnp.zeros((n // NS, d), jnp.float32)

    deg = _make_deg_kernel(n, ch)(src3, dst3, ones16, zeros16)
    ns, nd, h1 = _norm_mm_tc(deg, in_feat, W1)

    agg = _make_agg_kernel(n, d, ch)
    p1 = agg(h1, src3, dst3, zerosd)
    h2 = _post_pre_tc(p1, nd, b1, g1, bt1, ns, W2)
    p2 = agg(h2, src3, dst3, zerosd)
    t3 = _post_pre_nomm_tc(p2, nd, b2, g2, bt2, ns)
    p3 = agg(t3, src3, dst3, zerosd)
    return _final_tc(p3, nd, W3, b3)


# speedup vs baseline: 11.1277x; 11.1277x over previous
import functools

import jax
import jax.numpy as jnp
from jax import lax
from jax.experimental import pallas as pl
from jax.experimental.pallas import tpu as pltpu
from jax.experimental.pallas import tpu_sc as plsc

NC = 2
NS = 16
NW = NC * NS
K = 128


def _sc_mesh():
    return plsc.VectorSubcoreMesh(core_axis_name="c", subcore_axis_name="s")


def _make_deg_kernel(np_, ch):
    rows = np_ // NS

    @functools.partial(
        pl.kernel,
        mesh=_sc_mesh(),
        out_type=(jax.ShapeDtypeStruct((NC * np_,), jnp.float32),
                  jax.ShapeDtypeStruct((NC * np_,), jnp.float32)),
        scratch_types=[
            pltpu.VMEM((ch, K), jnp.int32),
            pltpu.VMEM((ch, K), jnp.int32),
            pltpu.VMEM((K,), jnp.float32),
            pltpu.VMEM_SHARED((np_,), jnp.float32),
            pltpu.VMEM_SHARED((np_,), jnp.float32),
        ],
    )
    def deg_kernel(src_h, dst_h, ones_h, z_h, do_h, di_h, src_v, dst_v,
                   ones_v, acc_o, acc_i):
        cid = lax.axis_index("c")
        sid = lax.axis_index("s")
        wid = sid * NC + cid
        r0 = pl.multiple_of(sid * rows, 128)
        pltpu.sync_copy(src_h.at[pl.ds(wid * ch, ch)], src_v)
        pltpu.sync_copy(dst_h.at[pl.ds(wid * ch, ch)], dst_v)
        pltpu.sync_copy(ones_h, ones_v)
        pltpu.sync_copy(z_h, acc_o.at[pl.ds(r0, rows)])
        pltpu.sync_copy(z_h, acc_i.at[pl.ds(r0, rows)])
        plsc.subcore_barrier()

        def body(j, carry):
            pltpu.sync_copy(ones_v, acc_o.at[src_v.at[j]], add=True)
            pltpu.sync_copy(ones_v, acc_i.at[dst_v.at[j]], add=True)
            return carry

        lax.fori_loop(0, ch, body, 0)
        plsc.subcore_barrier()
        o0 = pl.multiple_of(cid * np_ + r0, 128)
        pltpu.sync_copy(acc_o.at[pl.ds(r0, rows)], do_h.at[pl.ds(o0, rows)])
        pltpu.sync_copy(acc_i.at[pl.ds(r0, rows)], di_h.at[pl.ds(o0, rows)])

    return deg_kernel


def _make_agg_kernel(np_, d, ch):
    rows = np_ // NS

    @functools.partial(
        pl.kernel,
        mesh=_sc_mesh(),
        out_type=jax.ShapeDtypeStruct((NC, np_, d), jnp.float32),
        scratch_types=[
            pltpu.VMEM((ch, K), jnp.int32),
            pltpu.VMEM((2, K), jnp.int32),
            pltpu.VMEM((K, d), jnp.float32),
            pltpu.VMEM((K, d), jnp.float32),
            pltpu.VMEM_SHARED((np_, d), jnp.float32),
            pltpu.SemaphoreType.DMA,
            pltpu.SemaphoreType.DMA,
        ],
    )
    def agg_kernel(h_h, src_h, dst_h, z_h, out_h, src_v, dring,
                   rows_a, rows_b, acc, sem, semd):
        cid = lax.axis_index("c")
        sid = lax.axis_index("s")
        wid = sid * NC + cid
        r0 = pl.multiple_of(sid * rows, 128)
        b0 = wid * ch
        pltpu.sync_copy(src_h.at[pl.ds(b0, ch)], src_v)
        pltpu.sync_copy(z_h, acc.at[pl.ds(r0, rows)])
        plsc.subcore_barrier()

        pltpu.async_copy(h_h.at[src_v.at[0]], rows_a, sem).wait()
        pltpu.sync_copy(dst_h.at[b0], dring.at[0])

        def body2(i, carry):
            j = i * 2
            cp = pltpu.async_copy(h_h.at[src_v.at[j + 1]], rows_b, sem)
            cpd = pltpu.async_copy(dst_h.at[b0 + j + 1], dring.at[1], semd)
            pltpu.sync_copy(rows_a, acc.at[dring.at[0]], add=True)
            cp.wait()
            cpd.wait()
            cp = pltpu.async_copy(h_h.at[src_v.at[j + 2]], rows_a, sem)
            cpd = pltpu.async_copy(dst_h.at[b0 + j + 2], dring.at[0], semd)
            pltpu.sync_copy(rows_b, acc.at[dring.at[1]], add=True)
            cp.wait()
            cpd.wait()
            return carry

        lax.fori_loop(0, (ch - 2) // 2, body2, 0)
        cp = pltpu.async_copy(h_h.at[src_v.at[ch - 1]], rows_b, sem)
        cpd = pltpu.async_copy(dst_h.at[b0 + ch - 1], dring.at[1], semd)
        pltpu.sync_copy(rows_a, acc.at[dring.at[0]], add=True)
        cp.wait()
        cpd.wait()
        pltpu.sync_copy(rows_b, acc.at[dring.at[1]], add=True)

        plsc.subcore_barrier()
        pltpu.sync_copy(acc.at[pl.ds(r0, rows)],
                        out_h.at[cid, pl.ds(r0, rows)])

    return agg_kernel


def _norm_mm_tc(deg_o, deg_i, x, w, np_):
    n, d = x.shape
    h_dim = w.shape[1]

    def body(do_ref, di_ref, x_ref, w_ref, ns_ref, nd_ref, h_ref):
        dout = do_ref[0, :n] + do_ref[1, :n]
        din = di_ref[0, :n] + di_ref[1, :n]
        ns = jnp.where(dout > 0, lax.rsqrt(jnp.maximum(dout, 1.0)), 0.0)
        nd = jnp.where(din > 0, lax.rsqrt(jnp.maximum(din, 1.0)), 0.0)
        ns_ref[...] = ns
        nd_ref[...] = nd
        h_ref[:n] = jnp.dot(x_ref[...] * ns[:, None], w_ref[...],
                            preferred_element_type=jnp.float32)
        h_ref[n:] = jnp.zeros((np_ - n, h_dim), jnp.float32)

    return pl.pallas_call(
        body,
        out_shape=(
            jax.ShapeDtypeStruct((n,), jnp.float32),
            jax.ShapeDtypeStruct((n,), jnp.float32),
            jax.ShapeDtypeStruct((np_, h_dim), jnp.float32),
        ),
    )(deg_o, deg_i, x, w)


def _post_pre_tc(p, nd, b, g, bt, ns, w, np_):
    n = nd.shape[0]
    h_dim = w.shape[1]

    def body(p_ref, nd_ref, b_ref, g_ref, bt_ref, ns_ref, w_ref, h_ref):
        a = ((p_ref[0, :n] + p_ref[1, :n]) * nd_ref[...][:, None] + b_ref[...])
        mean = jnp.mean(a, axis=0)
        var = jnp.mean((a - mean) ** 2, axis=0)
        y = g_ref[...] * (a - mean) * lax.rsqrt(var + 1e-5) + bt_ref[...]
        y = jnp.maximum(y, 0.0)
        h_ref[:n] = jnp.dot(y * ns_ref[...][:, None], w_ref[...],
                            preferred_element_type=jnp.float32)
        h_ref[n:] = jnp.zeros((np_ - n, h_dim), jnp.float32)

    return pl.pallas_call(
        body,
        out_shape=jax.ShapeDtypeStruct((np_, h_dim), jnp.float32),
    )(p, nd, b, g, bt, ns, w)


def _post_pre_nomm_tc(p, nd, b, g, bt, ns, np_):
    n, d = nd.shape[0], p.shape[2]

    def body(p_ref, nd_ref, b_ref, g_ref, bt_ref, ns_ref, t_ref):
        a = ((p_ref[0, :n] + p_ref[1, :n]) * nd_ref[...][:, None] + b_ref[...])
        mean = jnp.mean(a, axis=0)
        var = jnp.mean((a - mean) ** 2, axis=0)
        y = g_ref[...] * (a - mean) * lax.rsqrt(var + 1e-5) + bt_ref[...]
        y = jnp.maximum(y, 0.0)
        t_ref[:n] = y * ns_ref[...][:, None]
        t_ref[n:] = jnp.zeros((np_ - n, d), jnp.float32)

    return pl.pallas_call(
        body,
        out_shape=jax.ShapeDtypeStruct((np_, d), jnp.float32),
    )(p, nd, b, g, bt, ns)


def _final_tc(p, nd, w, b):
    n = nd.shape[0]
    c = w.shape[1]

    def body(p_ref, nd_ref, w_ref, b_ref, o_ref):
        agg = p_ref[0, :n] + p_ref[1, :n]
        o_ref[...] = (jnp.dot(agg, w_ref[...],
                              preferred_element_type=jnp.float32)
                      * nd_ref[...][:, None] + b_ref[...])

    return pl.pallas_call(
        body,
        out_shape=jax.ShapeDtypeStruct((n, c), jnp.float32),
    )(p, nd, w, b)


def kernel(in_feat, edge_index, W1, b1, g1, bt1, W2, b2, g2, bt2, W3, b3):
    n, d = in_feat.shape
    e = edge_index.shape[1]
    np_ = ((n + NS * 128 - 1) // (NS * 128)) * (NS * 128)
    npad = np_ - n
    assert npad > 0
    blk = NW * K
    ep = ((e + blk - 1) // blk) * blk
    if (ep // blk) % 2:
        ep += blk
    ch = ep // blk
    pad = ep - e

    pad_idx = (n + jnp.arange(pad, dtype=jnp.int32) % npad)
    src = jnp.concatenate([edge_index[0], pad_idx]).reshape(ep // K, K)
    dst = jnp.concatenate([edge_index[1], pad_idx]).reshape(ep // K, K)
    ones_k = jnp.ones((K,), jnp.float32)
    zeros1 = jnp.zeros((np_ // NS,), jnp.float32)
    zerosd = jnp.zeros((np_ // NS, d), jnp.float32)

    deg_o, deg_i = _make_deg_kernel(np_, ch)(src, dst, ones_k, zeros1)
    deg_o = deg_o.reshape(NC, np_)
    deg_i = deg_i.reshape(NC, np_)
    ns, nd, h1 = _norm_mm_tc(deg_o, deg_i, in_feat, W1, np_)

    agg = _make_agg_kernel(np_, d, ch)
    p1 = agg(h1, src, dst, zerosd)
    h2 = _post_pre_tc(p1, nd, b1, g1, bt1, ns, W2, np_)
    p2 = agg(h2, src, dst, zerosd)
    t3 = _post_pre_nomm_tc(p2, nd, b2, g2, bt2, ns, np_)
    p3 = agg(t3, src, dst, zerosd)
    return _final_tc(p3, nd, W3, b3)


# speedup vs baseline: 11.2758x; 1.0133x over previous
import functools

import jax
import jax.numpy as jnp
from jax import lax
from jax.experimental import pallas as pl
from jax.experimental.pallas import tpu as pltpu
from jax.experimental.pallas import tpu_sc as plsc

NC = 2
NS = 16
NW = NC * NS
K = 128


def _sc_mesh():
    return plsc.VectorSubcoreMesh(core_axis_name="c", subcore_axis_name="s")


def _make_deg_kernel(np_, ch):
    rows = np_ // NS

    @functools.partial(
        pl.kernel,
        mesh=_sc_mesh(),
        out_type=(jax.ShapeDtypeStruct((NC * np_,), jnp.float32),
                  jax.ShapeDtypeStruct((NC * np_,), jnp.float32)),
        scratch_types=[
            pltpu.VMEM((ch, K), jnp.int32),
            pltpu.VMEM((ch, K), jnp.int32),
            pltpu.VMEM((K,), jnp.float32),
            pltpu.VMEM_SHARED((np_,), jnp.float32),
            pltpu.VMEM_SHARED((np_,), jnp.float32),
            pltpu.SemaphoreType.DMA,
        ],
    )
    def deg_kernel(src_h, dst_h, ones_h, z_h, do_h, di_h, src_v, dst_v,
                   ones_v, acc_o, acc_i, sems):
        cid = lax.axis_index("c")
        sid = lax.axis_index("s")
        wid = sid * NC + cid
        r0 = pl.multiple_of(sid * rows, 128)
        pltpu.sync_copy(src_h.at[pl.ds(wid * ch, ch)], src_v)
        pltpu.sync_copy(dst_h.at[pl.ds(wid * ch, ch)], dst_v)
        pltpu.sync_copy(ones_h, ones_v)
        pltpu.sync_copy(z_h, acc_o.at[pl.ds(r0, rows)])
        pltpu.sync_copy(z_h, acc_i.at[pl.ds(r0, rows)])
        plsc.subcore_barrier()

        def body(j, carry):
            cp = pltpu.async_copy(ones_v, acc_o.at[src_v.at[j]], sems,
                                  add=True)
            pltpu.sync_copy(ones_v, acc_i.at[dst_v.at[j]], add=True)
            cp.wait()
            return carry

        lax.fori_loop(0, ch, body, 0)
        plsc.subcore_barrier()
        o0 = pl.multiple_of(cid * np_ + r0, 128)
        pltpu.sync_copy(acc_o.at[pl.ds(r0, rows)], do_h.at[pl.ds(o0, rows)])
        pltpu.sync_copy(acc_i.at[pl.ds(r0, rows)], di_h.at[pl.ds(o0, rows)])

    return deg_kernel


def _make_agg_kernel(np_, d, ch):
    rows = np_ // NS

    @functools.partial(
        pl.kernel,
        mesh=_sc_mesh(),
        out_type=jax.ShapeDtypeStruct((NC, np_, d), jnp.float32),
        scratch_types=[
            pltpu.VMEM((ch, K), jnp.int32),
            pltpu.VMEM((2, K), jnp.int32),
            pltpu.VMEM((K, d), jnp.float32),
            pltpu.VMEM((K, d), jnp.float32),
            pltpu.VMEM_SHARED((np_, d), jnp.float32),
            pltpu.SemaphoreType.DMA,
            pltpu.SemaphoreType.DMA,
        ],
    )
    def agg_kernel(h_h, src_h, dst_h, z_h, out_h, src_v, dring,
                   rows_a, rows_b, acc, sem, semd):
        cid = lax.axis_index("c")
        sid = lax.axis_index("s")
        wid = sid * NC + cid
        r0 = pl.multiple_of(sid * rows, 128)
        b0 = wid * ch
        cpz = pltpu.async_copy(z_h, acc.at[pl.ds(r0, rows)], semd)
        pltpu.sync_copy(src_h.at[pl.ds(b0, ch)], src_v)
        cpz.wait()
        plsc.subcore_barrier()

        pltpu.async_copy(h_h.at[src_v.at[0]], rows_a, sem).wait()
        pltpu.sync_copy(dst_h.at[b0], dring.at[0])

        def body2(i, carry):
            j = i * 2
            cp = pltpu.async_copy(h_h.at[src_v.at[j + 1]], rows_b, sem)
            cpd = pltpu.async_copy(dst_h.at[b0 + j + 1], dring.at[1], semd)
            pltpu.sync_copy(rows_a, acc.at[dring.at[0]], add=True)
            cp.wait()
            cpd.wait()
            cp = pltpu.async_copy(h_h.at[src_v.at[j + 2]], rows_a, sem)
            cpd = pltpu.async_copy(dst_h.at[b0 + j + 2], dring.at[0], semd)
            pltpu.sync_copy(rows_b, acc.at[dring.at[1]], add=True)
            cp.wait()
            cpd.wait()
            return carry

        lax.fori_loop(0, (ch - 2) // 2, body2, 0)
        cp = pltpu.async_copy(h_h.at[src_v.at[ch - 1]], rows_b, sem)
        cpd = pltpu.async_copy(dst_h.at[b0 + ch - 1], dring.at[1], semd)
        pltpu.sync_copy(rows_a, acc.at[dring.at[0]], add=True)
        cp.wait()
        cpd.wait()
        pltpu.sync_copy(rows_b, acc.at[dring.at[1]], add=True)

        plsc.subcore_barrier()
        pltpu.sync_copy(acc.at[pl.ds(r0, rows)],
                        out_h.at[cid, pl.ds(r0, rows)])

    return agg_kernel


def _mm_tc(x, w):
    n = x.shape[0]
    h_dim = w.shape[1]

    def body(x_ref, w_ref, m_ref):
        m_ref[...] = jnp.dot(x_ref[...], w_ref[...],
                             preferred_element_type=jnp.float32)

    return pl.pallas_call(
        body,
        out_shape=jax.ShapeDtypeStruct((n, h_dim), jnp.float32),
    )(x, w)


def _norm_scale_tc(deg_o, deg_i, m, np_):
    n, h_dim = m.shape

    def body(do_ref, di_ref, m_ref, ns_ref, nd_ref, h_ref):
        dout = do_ref[0, :n] + do_ref[1, :n]
        din = di_ref[0, :n] + di_ref[1, :n]
        ns = jnp.where(dout > 0, lax.rsqrt(jnp.maximum(dout, 1.0)), 0.0)
        nd = jnp.where(din > 0, lax.rsqrt(jnp.maximum(din, 1.0)), 0.0)
        ns_ref[...] = ns
        nd_ref[...] = nd
        h_ref[:n] = m_ref[...] * ns[:, None]
        h_ref[n:] = jnp.zeros((np_ - n, h_dim), jnp.float32)

    return pl.pallas_call(
        body,
        out_shape=(
            jax.ShapeDtypeStruct((n,), jnp.float32),
            jax.ShapeDtypeStruct((n,), jnp.float32),
            jax.ShapeDtypeStruct((np_, h_dim), jnp.float32),
        ),
    )(deg_o, deg_i, m)


def _post_pre_tc(p, nd, b, g, bt, ns, w, np_):
    n = nd.shape[0]
    h_dim = w.shape[1]

    def body(p_ref, nd_ref, b_ref, g_ref, bt_ref, ns_ref, w_ref, h_ref):
        a = ((p_ref[0, :n] + p_ref[1, :n]) * nd_ref[...][:, None] + b_ref[...])
        mean = jnp.mean(a, axis=0)
        var = jnp.mean((a - mean) ** 2, axis=0)
        y = g_ref[...] * (a - mean) * lax.rsqrt(var + 1e-5) + bt_ref[...]
        y = jnp.maximum(y, 0.0)
        h_ref[:n] = jnp.dot(y * ns_ref[...][:, None], w_ref[...],
                            preferred_element_type=jnp.float32)
        h_ref[n:] = jnp.zeros((np_ - n, h_dim), jnp.float32)

    return pl.pallas_call(
        body,
        out_shape=jax.ShapeDtypeStruct((np_, h_dim), jnp.float32),
    )(p, nd, b, g, bt, ns, w)


def _post_pre_nomm_tc(p, nd, b, g, bt, ns, np_):
    n, d = nd.shape[0], p.shape[2]

    def body(p_ref, nd_ref, b_ref, g_ref, bt_ref, ns_ref, t_ref):
        a = ((p_ref[0, :n] + p_ref[1, :n]) * nd_ref[...][:, None] + b_ref[...])
        mean = jnp.mean(a, axis=0)
        var = jnp.mean((a - mean) ** 2, axis=0)
        y = g_ref[...] * (a - mean) * lax.rsqrt(var + 1e-5) + bt_ref[...]
        y = jnp.maximum(y, 0.0)
        t_ref[:n] = y * ns_ref[...][:, None]
        t_ref[n:] = jnp.zeros((np_ - n, d), jnp.float32)

    return pl.pallas_call(
        body,
        out_shape=jax.ShapeDtypeStruct((np_, d), jnp.float32),
    )(p, nd, b, g, bt, ns)


def _final_tc(p, nd, w, b):
    n = nd.shape[0]
    c = w.shape[1]

    def body(p_ref, nd_ref, w_ref, b_ref, o_ref):
        agg = p_ref[0, :n] + p_ref[1, :n]
        o_ref[...] = (jnp.dot(agg, w_ref[...],
                              preferred_element_type=jnp.float32)
                      * nd_ref[...][:, None] + b_ref[...])

    return pl.pallas_call(
        body,
        out_shape=jax.ShapeDtypeStruct((n, c), jnp.float32),
    )(p, nd, w, b)


def kernel(in_feat, edge_index, W1, b1, g1, bt1, W2, b2, g2, bt2, W3, b3):
    n, d = in_feat.shape
    e = edge_index.shape[1]
    np_ = ((n + NS * 128 - 1) // (NS * 128)) * (NS * 128)
    npad = np_ - n
    assert npad > 0
    blk = NW * K
    ep = ((e + blk - 1) // blk) * blk
    if (ep // blk) % 2:
        ep += blk
    ch = ep // blk
    pad = ep - e

    pad_idx = (n + jnp.arange(pad, dtype=jnp.int32) % npad)
    src = jnp.concatenate([edge_index[0], pad_idx]).reshape(ep // K, K)
    dst = jnp.concatenate([edge_index[1], pad_idx]).reshape(ep // K, K)
    ones_k = jnp.ones((K,), jnp.float32)
    zeros1 = jnp.zeros((np_ // NS,), jnp.float32)
    zerosd = jnp.zeros((np_ // NS, d), jnp.float32)

    m1 = _mm_tc(in_feat, W1)
    deg_o, deg_i = _make_deg_kernel(np_, ch)(src, dst, ones_k, zeros1)
    deg_o = deg_o.reshape(NC, np_)
    deg_i = deg_i.reshape(NC, np_)
    ns, nd, h1 = _norm_scale_tc(deg_o, deg_i, m1, np_)

    agg = _make_agg_kernel(np_, d, ch)
    p1 = agg(h1, src, dst, zerosd)
    h2 = _post_pre_tc(p1, nd, b1, g1, bt1, ns, W2, np_)
    p2 = agg(h2, src, dst, zerosd)
    t3 = _post_pre_nomm_tc(p2, nd, b2, g2, bt2, ns, np_)
    p3 = agg(t3, src, dst, zerosd)
    return _final_tc(p3, nd, W3, b3)


# speedup vs baseline: 11.4407x; 1.0146x over previous
import functools

import jax
import jax.numpy as jnp
from jax import lax
from jax.experimental import pallas as pl
from jax.experimental.pallas import tpu as pltpu
from jax.experimental.pallas import tpu_sc as plsc

NC = 2
NS = 16
NW = NC * NS
K = 128


def _sc_mesh():
    return plsc.VectorSubcoreMesh(core_axis_name="c", subcore_axis_name="s")


def _make_deg_kernel(np_, ch):
    rows = np_ // NS

    @functools.partial(
        pl.kernel,
        mesh=_sc_mesh(),
        out_type=(jax.ShapeDtypeStruct((NC * np_,), jnp.float32),
                  jax.ShapeDtypeStruct((NC * np_,), jnp.float32)),
        scratch_types=[
            pltpu.VMEM((ch, K), jnp.int32),
            pltpu.VMEM((ch, K), jnp.int32),
            pltpu.VMEM((K,), jnp.float32),
            pltpu.VMEM_SHARED((np_,), jnp.float32),
            pltpu.VMEM_SHARED((np_,), jnp.float32),
            pltpu.SemaphoreType.DMA,
        ],
    )
    def deg_kernel(src_h, dst_h, ones_h, z_h, do_h, di_h, src_v, dst_v,
                   ones_v, acc_o, acc_i, sems):
        cid = lax.axis_index("c")
        sid = lax.axis_index("s")
        wid = sid * NC + cid
        r0 = pl.multiple_of(sid * rows, 128)
        pltpu.sync_copy(src_h.at[pl.ds(wid * ch, ch)], src_v)
        pltpu.sync_copy(dst_h.at[pl.ds(wid * ch, ch)], dst_v)
        pltpu.sync_copy(ones_h, ones_v)
        pltpu.sync_copy(z_h, acc_o.at[pl.ds(r0, rows)])
        pltpu.sync_copy(z_h, acc_i.at[pl.ds(r0, rows)])
        plsc.subcore_barrier()

        def body(j, carry):
            cp = pltpu.async_copy(ones_v, acc_o.at[src_v.at[j]], sems,
                                  add=True)
            pltpu.sync_copy(ones_v, acc_i.at[dst_v.at[j]], add=True)
            cp.wait()
            return carry

        lax.fori_loop(0, ch, body, 0)
        plsc.subcore_barrier()
        o0 = pl.multiple_of(cid * np_ + r0, 128)
        pltpu.sync_copy(acc_o.at[pl.ds(r0, rows)], do_h.at[pl.ds(o0, rows)])
        pltpu.sync_copy(acc_i.at[pl.ds(r0, rows)], di_h.at[pl.ds(o0, rows)])

    return deg_kernel


def _make_agg_kernel(np_, d, ch):
    rows = np_ // NS

    @functools.partial(
        pl.kernel,
        mesh=_sc_mesh(),
        out_type=jax.ShapeDtypeStruct((NC, np_, d), jnp.float32),
        scratch_types=[
            pltpu.VMEM((ch, K), jnp.int32),
            pltpu.VMEM((2, K), jnp.int32),
            pltpu.VMEM((K, d), jnp.float32),
            pltpu.VMEM((K, d), jnp.float32),
            pltpu.VMEM_SHARED((np_, d), jnp.float32),
            pltpu.SemaphoreType.DMA,
            pltpu.SemaphoreType.DMA,
        ],
    )
    def agg_kernel(h_h, src_h, dst_h, z_h, out_h, src_v, dring,
                   rows_a, rows_b, acc, sem, semd):
        cid = lax.axis_index("c")
        sid = lax.axis_index("s")
        wid = sid * NC + cid
        r0 = pl.multiple_of(sid * rows, 128)
        b0 = wid * ch
        cpz = pltpu.async_copy(z_h, acc.at[pl.ds(r0, rows)], semd)
        pltpu.sync_copy(src_h.at[pl.ds(b0, ch)], src_v)
        cp0 = pltpu.async_copy(h_h.at[src_v.at[0]], rows_a, sem)
        pltpu.sync_copy(dst_h.at[b0], dring.at[0])
        cpz.wait()
        plsc.subcore_barrier()

        cp0.wait()

        def body2(i, carry):
            j = i * 2
            cp = pltpu.async_copy(h_h.at[src_v.at[j + 1]], rows_b, sem)
            cpd = pltpu.async_copy(dst_h.at[b0 + j + 1], dring.at[1], semd)
            pltpu.sync_copy(rows_a, acc.at[dring.at[0]], add=True)
            cp.wait()
            cpd.wait()
            cp = pltpu.async_copy(h_h.at[src_v.at[j + 2]], rows_a, sem)
            cpd = pltpu.async_copy(dst_h.at[b0 + j + 2], dring.at[0], semd)
            pltpu.sync_copy(rows_b, acc.at[dring.at[1]], add=True)
            cp.wait()
            cpd.wait()
            return carry

        lax.fori_loop(0, (ch - 2) // 2, body2, 0)
        cp = pltpu.async_copy(h_h.at[src_v.at[ch - 1]], rows_b, sem)
        cpd = pltpu.async_copy(dst_h.at[b0 + ch - 1], dring.at[1], semd)
        pltpu.sync_copy(rows_a, acc.at[dring.at[0]], add=True)
        cp.wait()
        cpd.wait()
        pltpu.sync_copy(rows_b, acc.at[dring.at[1]], add=True)

        plsc.subcore_barrier()
        pltpu.sync_copy(acc.at[pl.ds(r0, rows)],
                        out_h.at[cid, pl.ds(r0, rows)])

    return agg_kernel


def _mm_tc(x, w):
    n = x.shape[0]
    h_dim = w.shape[1]

    def body(x_ref, w_ref, m_ref):
        m_ref[...] = jnp.dot(x_ref[...], w_ref[...],
                             preferred_element_type=jnp.float32)

    return pl.pallas_call(
        body,
        out_shape=jax.ShapeDtypeStruct((n, h_dim), jnp.float32),
    )(x, w)


def _norm_scale_tc(deg_o, deg_i, m, np_):
    n, h_dim = m.shape

    def body(do_ref, di_ref, m_ref, ns_ref, nd_ref, h_ref):
        dout = do_ref[0, :n] + do_ref[1, :n]
        din = di_ref[0, :n] + di_ref[1, :n]
        ns = jnp.where(dout > 0, lax.rsqrt(jnp.maximum(dout, 1.0)), 0.0)
        nd = jnp.where(din > 0, lax.rsqrt(jnp.maximum(din, 1.0)), 0.0)
        ns_ref[...] = ns
        nd_ref[...] = nd
        h_ref[:n] = m_ref[...] * ns[:, None]
        h_ref[n:] = jnp.zeros((np_ - n, h_dim), jnp.float32)

    return pl.pallas_call(
        body,
        out_shape=(
            jax.ShapeDtypeStruct((n,), jnp.float32),
            jax.ShapeDtypeStruct((n,), jnp.float32),
            jax.ShapeDtypeStruct((np_, h_dim), jnp.float32),
        ),
    )(deg_o, deg_i, m)


def _post_pre_tc(p, nd, b, g, bt, ns, w, np_):
    n = nd.shape[0]
    h_dim = w.shape[1]

    def body(p_ref, nd_ref, b_ref, g_ref, bt_ref, ns_ref, w_ref, h_ref):
        a = ((p_ref[0, :n] + p_ref[1, :n]) * nd_ref[...][:, None] + b_ref[...])
        mean = jnp.mean(a, axis=0)
        var = jnp.mean((a - mean) ** 2, axis=0)
        y = g_ref[...] * (a - mean) * lax.rsqrt(var + 1e-5) + bt_ref[...]
        y = jnp.maximum(y, 0.0)
        h_ref[:n] = jnp.dot(y * ns_ref[...][:, None], w_ref[...],
                            preferred_element_type=jnp.float32)
        h_ref[n:] = jnp.zeros((np_ - n, h_dim), jnp.float32)

    return pl.pallas_call(
        body,
        out_shape=jax.ShapeDtypeStruct((np_, h_dim), jnp.float32),
    )(p, nd, b, g, bt, ns, w)


def _post_pre_nomm_tc(p, nd, b, g, bt, ns, np_):
    n, d = nd.shape[0], p.shape[2]

    def body(p_ref, nd_ref, b_ref, g_ref, bt_ref, ns_ref, t_ref):
        a = ((p_ref[0, :n] + p_ref[1, :n]) * nd_ref[...][:, None] + b_ref[...])
        mean = jnp.mean(a, axis=0)
        var = jnp.mean((a - mean) ** 2, axis=0)
        y = g_ref[...] * (a - mean) * lax.rsqrt(var + 1e-5) + bt_ref[...]
        y = jnp.maximum(y, 0.0)
        t_ref[:n] = y * ns_ref[...][:, None]
        t_ref[n:] = jnp.zeros((np_ - n, d), jnp.float32)

    return pl.pallas_call(
        body,
        out_shape=jax.ShapeDtypeStruct((np_, d), jnp.float32),
    )(p, nd, b, g, bt, ns)


def _final_tc(p, nd, w, b):
    n = nd.shape[0]
    c = w.shape[1]

    def body(p_ref, nd_ref, w_ref, b_ref, o_ref):
        agg = p_ref[0, :n] + p_ref[1, :n]
        o_ref[...] = (jnp.dot(agg, w_ref[...],
                              preferred_element_type=jnp.float32)
                      * nd_ref[...][:, None] + b_ref[...])

    return pl.pallas_call(
        body,
        out_shape=jax.ShapeDtypeStruct((n, c), jnp.float32),
    )(p, nd, w, b)


def kernel(in_feat, edge_index, W1, b1, g1, bt1, W2, b2, g2, bt2, W3, b3):
    n, d = in_feat.shape
    e = edge_index.shape[1]
    np_ = ((n + NS * 128 - 1) // (NS * 128)) * (NS * 128)
    npad = np_ - n
    assert npad > 0
    blk = NW * K
    ep = ((e + blk - 1) // blk) * blk
    if (ep // blk) % 2:
        ep += blk
    ch = ep // blk
    pad = ep - e

    pad_idx = (n + jnp.arange(pad, dtype=jnp.int32) % npad)
    src = jnp.concatenate([edge_index[0], pad_idx]).reshape(ep // K, K)
    dst = jnp.concatenate([edge_index[1], pad_idx]).reshape(ep // K, K)
    ones_k = jnp.ones((K,), jnp.float32)
    zeros1 = jnp.zeros((np_ // NS,), jnp.float32)
    zerosd = jnp.zeros((np_ // NS, d), jnp.float32)

    m1 = _mm_tc(in_feat, W1)
    deg_o, deg_i = _make_deg_kernel(np_, ch)(src, dst, ones_k, zeros1)
    deg_o = deg_o.reshape(NC, np_)
    deg_i = deg_i.reshape(NC, np_)
    ns, nd, h1 = _norm_scale_tc(deg_o, deg_i, m1, np_)

    agg = _make_agg_kernel(np_, d, ch)
    p1 = agg(h1, src, dst, zerosd)
    h2 = _post_pre_tc(p1, nd, b1, g1, bt1, ns, W2, np_)
    p2 = agg(h2, src, dst, zerosd)
    t3 = _post_pre_nomm_tc(p2, nd, b2, g2, bt2, ns, np_)
    p3 = agg(t3, src, dst, zerosd)
    return _final_tc(p3, nd, W3, b3)
